# baseline (speedup 1.0000x reference)
import functools

import jax
import jax.numpy as jnp
from jax import lax
from jax.experimental import pallas as pl
from jax.experimental.pallas import tpu as pltpu
from jax.experimental.pallas import tpu_sc as plsc

_B, _T, _D, _E = 4, 8192, 1024, 64
_T_SC = 2048
_T_TC = _T - _T_SC
_NC, _NS = 2, 16
_NW = _NC * _NS
_RPW = _T_SC // _NW
_CH = 16
_SC_NBUF = 4
_CPB = _RPW // _CH
_NCHUNK = _B * _CPB
_LANES = 16

_CHR = 512
_TCPB = _T_TC // _CHR
_TC_NCH = _B * _TCPB
_TC_NBUF = 8
_AW = 32


def _pool_body(x_hbm, out_hbm, b0, b1, b2, b3, acc, s0, s1, s2, s3):
    bufs = (b0, b1, b2, b3)
    sems = (s0, s1, s2, s3)
    wid = lax.axis_index("s") * _NC + lax.axis_index("c")

    def chunk_src(c):
        b = c // _CPB
        sub = lax.rem(c, _CPB)
        row0 = b * _T + wid * _RPW + sub * _CH
        return x_hbm.at[pl.ds(row0, _CH)]

    def zero_body(z, _):
        acc[0, pl.ds(z * _LANES, _LANES)] = jnp.zeros((_LANES,), jnp.float32)
        acc[1, pl.ds(z * _LANES, _LANES)] = jnp.zeros((_LANES,), jnp.float32)
        acc[2, pl.ds(z * _LANES, _LANES)] = jnp.zeros((_LANES,), jnp.float32)
        acc[3, pl.ds(z * _LANES, _LANES)] = jnp.zeros((_LANES,), jnp.float32)
        return 0
    lax.fori_loop(0, _D // _LANES, zero_body, 0)

    for k in range(_SC_NBUF):
        pltpu.make_async_copy(chunk_src(k), bufs[k], sems[k]).start()

    def accumulate(c, buf):
        b = c // _CPB

        def d_body(d, _):
            a = acc[b, pl.ds(d * _LANES, _LANES)]
            vs = [buf[j, pl.ds(d * _LANES, _LANES)] for j in range(_CH)]
            while len(vs) > 1:
                vs = [vs[i] + vs[i + 1] for i in range(0, len(vs) - 1, 2)] + (
                    [vs[-1]] if len(vs) % 2 else [])
            acc[b, pl.ds(d * _LANES, _LANES)] = a + vs[0]
            return 0
        lax.fori_loop(0, _D // _LANES, d_body, 0)

    def ring_body(i, _):
        for k in range(_SC_NBUF):
            c = i * _SC_NBUF + k
            pltpu.make_async_copy(chunk_src(c), bufs[k], sems[k]).wait()
            accumulate(c, bufs[k])

            @pl.when(c + _SC_NBUF < _NCHUNK)
            def _():
                pltpu.make_async_copy(chunk_src(c + _SC_NBUF), bufs[k],
                                      sems[k]).start()
        return 0
    lax.fori_loop(0, _NCHUNK // _SC_NBUF, ring_body, 0)

    pltpu.sync_copy(acc, out_hbm.at[wid])


@functools.cache
def _pool():
    return pl.kernel(
        _pool_body,
        out_type=jax.ShapeDtypeStruct((_NW, _B, _D), jnp.float32),
        mesh=plsc.VectorSubcoreMesh(core_axis_name="c", subcore_axis_name="s",
                                    num_cores=_NC, num_subcores=_NS),
        scratch_types=(
            [pltpu.VMEM((_CH, _D), jnp.float32) for _ in range(_SC_NBUF)]
            + [pltpu.VMEM((_B, _D), jnp.float32)]
            + [pltpu.SemaphoreType.DMA for _ in range(_SC_NBUF)]
        ),
    )


def _tc_body(x_hbm, out_ref,
             b0, b1, b2, b3, b4, b5, b6, b7, acc_ref,
             s0, s1, s2, s3, s4, s5, s6, s7):
    bufs = (b0, b1, b2, b3, b4, b5, b6, b7)
    sems = (s0, s1, s2, s3, s4, s5, s6, s7)

    def dma(c, k):
        b, sub = c // _TCPB, c % _TCPB
        row0 = b * _T + _T_SC + sub * _CHR
        return pltpu.make_async_copy(
            x_hbm.at[pl.ds(row0, _CHR)], bufs[k], sems[k])

    for k in range(_TC_NBUF):
        dma(k, k).start()

    for c in range(_TC_NCH):
        k = c % _TC_NBUF
        dma(c, k).wait()
        buf = bufs[k]
        b = c // _TCPB
        s = buf[0:_AW]
        for i in range(1, _CHR // _AW):
            s = s + buf[i * _AW:(i + 1) * _AW]
        if c % _TCPB == 0:
            acc_ref[b] = s
        else:
            acc_ref[b] += s
        if c + _TC_NBUF < _TC_NCH:
            dma(c + _TC_NBUF, k).start()

    out_ref[...] = jnp.sum(acc_ref[...], axis=1)


def _tc_reduce(xf):
    return pl.pallas_call(
        _tc_body,
        in_specs=[pl.BlockSpec(memory_space=pl.ANY)],
        out_shape=jax.ShapeDtypeStruct((_B, _D), jnp.float32),
        scratch_shapes=(
            [pltpu.VMEM((_CHR, _D), jnp.float32) for _ in range(_TC_NBUF)]
            + [pltpu.VMEM((_B, _AW, _D), jnp.float32)]
            + [pltpu.SemaphoreType.DMA for _ in range(_TC_NBUF)]
        ),
    )(xf)


def _finale_body(p_ref, ptc_ref, w_ref, b_ref, idx_ref, probs_ref):
    pooled = (jnp.sum(p_ref[...], axis=0) + ptc_ref[...]) * (1.0 / _T)
    logits = lax.dot_general(
        pooled, w_ref[...], (((1,), (1,)), ((), ())),
        preferred_element_type=jnp.float32) + b_ref[...][None, :]
    m = jnp.max(logits, axis=-1, keepdims=True)
    e = jnp.exp(logits - m)
    probs = e / jnp.sum(e, axis=-1, keepdims=True)
    probs_ref[...] = probs
    idx_ref[...] = jnp.argmax(probs, axis=-1).astype(jnp.int32)


def _finale(partials, partial_tc, W, expert_bias):
    return pl.pallas_call(
        _finale_body,
        out_shape=(jax.ShapeDtypeStruct((_B,), jnp.int32),
                   jax.ShapeDtypeStruct((_B, _E), jnp.float32)),
    )(partials, partial_tc, W, expert_bias)


def kernel(x, W, expert_bias):
    xf = x.reshape(_B * _T, _D)
    partials = _pool()(xf)
    partial_tc = _tc_reduce(xf)
    return _finale(partials, partial_tc, W, expert_bias)
